# baseline (speedup 1.0000x reference)
import functools

import jax
import jax.numpy as jnp
from jax.experimental import pallas as pl
from jax.experimental.pallas import tpu as pltpu

EPS = 1e-5
B = 400
S = 8 * B


def _scatter_kernel(slot_hbm, zeros_any, src_blk, ebuf, idx_smem, sbuf, sem_i,
                    sem_s, *, relu_src, dump):
    b = pl.program_id(0)
    cp = pltpu.make_async_copy(slot_hbm.at[b], idx_smem, sem_i)
    cp.start()
    cp.wait()

    v = src_blk[...]
    if relu_src:
        v = jnp.maximum(v, 0.0)
    sbuf[...] = v.astype(jnp.bfloat16)

    def issue(i, carry):
        c = carry
        for kk in range(8):
            d = idx_smem[kk * B + i]
            ok = d < dump

            @pl.when(ok)
            def _():
                pltpu.make_async_copy(
                    sbuf.at[i],
                    ebuf.at[d],
                    sem_s,
                ).start(priority=kk % 2)

            c = c + jnp.where(ok, 1, 0)
        return c

    n_issued = jax.lax.fori_loop(0, B, issue, 0)

    @pl.when(n_issued > 0)
    def _wait_all():
        pltpu.make_async_copy(
            ebuf.at[pl.ds(0, n_issued)],
            ebuf.at[pl.ds(0, n_issued)],
            sem_s,
        ).wait()


def _scatter(slot_tbl, src3, nblk_src, relu_src):
    ztotal = (nblk_src + 1) * S
    zeros = jnp.zeros((ztotal, 2, 128), jnp.bfloat16)
    kern = functools.partial(_scatter_kernel, relu_src=relu_src,
                             dump=nblk_src * S)
    return pl.pallas_call(
        kern,
        grid=(nblk_src + 1,),
        in_specs=[
            pl.BlockSpec(memory_space=pl.ANY),
            pl.BlockSpec(memory_space=pl.ANY),
            pl.BlockSpec((B, 2, 128), lambda b: (b, 0, 0)),
        ],
        out_specs=pl.BlockSpec(memory_space=pl.ANY),
        out_shape=jax.ShapeDtypeStruct((ztotal, 2, 128), jnp.bfloat16),
        input_output_aliases={1: 0},
        scratch_shapes=[
            pltpu.SMEM((S,), jnp.int32),
            pltpu.VMEM((B, 2, 128), jnp.bfloat16),
            pltpu.SemaphoreType.DMA,
            pltpu.SemaphoreType.DMA,
        ],
        compiler_params=pltpu.CompilerParams(
            dimension_semantics=("parallel",),
        ),
    )(slot_tbl, zeros, src3)


def _compute_kernel(edge_blk, center_blk, res_blk, w8, wc, pw, sc, bi, out,
                    *, nblk, relu_center, relu_out, add_residual, out_dtype):
    b = pl.program_id(0)

    @pl.when(b < nblk)
    def _body():
        ctr = center_blk[...]
        if relu_center:
            ctr = jnp.maximum(ctr, 0.0)
        acc = ctr.astype(jnp.float32) * wc[...]
        for kk in range(8):
            g = edge_blk[kk * B:(kk + 1) * B, :, :].astype(jnp.float32)
            acc = acc + g * w8[kk:kk + 1, :, :]

        l0 = acc[:, 0, :].astype(jnp.bfloat16)
        l1 = acc[:, 1, :].astype(jnp.bfloat16)
        mm = (jnp.dot(l0, pw[:128, :], preferred_element_type=jnp.float32) +
              jnp.dot(l1, pw[128:, :], preferred_element_type=jnp.float32))
        h = mm * sc[...] + bi[...]
        if relu_out:
            h = jnp.maximum(h, 0.0)
        if add_residual:
            h = h + res_blk[...]
        out[...] = h.astype(out_dtype)

    if nblk < pl.num_programs(0):
        @pl.when(b >= nblk)
        def _zero_tail():
            out[...] = jnp.zeros(out.shape, out.dtype)


def _compute(edge_buf, center3, residual, w83, wc3, pw_bf16, sc, bi, *,
             n_out_rows, nblk, grid, relu_center, relu_out, add_residual,
             out_dtype):
    kern = functools.partial(
        _compute_kernel, nblk=nblk, relu_center=relu_center,
        relu_out=relu_out, add_residual=add_residual, out_dtype=out_dtype)
    blk = lambda b: (b, 0)
    blk3 = lambda b: (b, 0, 0)
    zero = lambda b: (0, 0)
    zero3 = lambda b: (0, 0, 0)
    if not add_residual:
        res_spec = pl.BlockSpec((1, 256), zero)
        residual = sc
    else:
        res_spec = pl.BlockSpec((B, 256), blk)
    return pl.pallas_call(
        kern,
        grid=(grid,),
        in_specs=[
            pl.BlockSpec((S, 2, 128), blk3),
            pl.BlockSpec((B, 2, 128), blk3),
            res_spec,
            pl.BlockSpec((8, 2, 128), zero3),
            pl.BlockSpec((1, 2, 128), zero3),
            pl.BlockSpec((256, 256), zero),
            pl.BlockSpec((1, 256), zero),
            pl.BlockSpec((1, 256), zero),
        ],
        out_specs=pl.BlockSpec((B, 256), blk),
        out_shape=jax.ShapeDtypeStruct((n_out_rows, 256), out_dtype),
        compiler_params=pltpu.CompilerParams(
            dimension_semantics=("parallel",),
        ),
    )(edge_buf, center3, residual, w83, wc3, pw_bf16, sc, bi)


def kernel(x, nbr_idx, dw_w1, pw_w1, bn1_g, bn1_b, bn1_m, bn1_v,
           dw_w2, pw_w2, bn2_g, bn2_b, bn2_m, bn2_v):
    n, c = x.shape
    assert c == 256 and n % B == 0
    nblk = n // B
    dump = nblk * S

    idx32 = nbr_idx.astype(jnp.int32)
    sel = jnp.concatenate([idx32[:4], idx32[5:]], axis=0)
    rev = sel[::-1]
    colid = jax.lax.broadcasted_iota(jnp.int32, (8, n), 1)
    dump_slot = dump + (colid % S)
    slot = jnp.where(rev >= 0,
                     (rev // B) * S + jnp.arange(8, dtype=jnp.int32)[:, None] * B
                     + (rev % B),
                     dump_slot)
    slot = jnp.pad(slot, ((0, 0), (0, B)), constant_values=dump)
    slot_tbl = slot.reshape(8, nblk + 1, B).transpose(1, 0, 2)
    slot_tbl = slot_tbl.reshape(nblk + 1, S)

    x3 = x.reshape(n, 2, 128)

    s1 = (bn1_g * jax.lax.rsqrt(bn1_v + EPS)).reshape(1, c)
    o1 = (bn1_b - bn1_m * s1[0]).reshape(1, c)
    s2 = (bn2_g * jax.lax.rsqrt(bn2_v + EPS)).reshape(1, c)
    o2 = (bn2_b - bn2_m * s2[0]).reshape(1, c)

    w8_1 = jnp.concatenate([dw_w1[:4], dw_w1[5:]], axis=0).reshape(8, 2, 128)
    wc_1 = dw_w1[4:5].reshape(1, 2, 128)
    w8_2 = jnp.concatenate([dw_w2[:4], dw_w2[5:]], axis=0).reshape(8, 2, 128)
    wc_2 = dw_w2[4:5].reshape(1, 2, 128)

    pw1b = pw_w1.astype(jnp.bfloat16)
    pw2b = pw_w2.astype(jnp.bfloat16)

    e1 = _scatter(slot_tbl, x3, nblk, relu_src=True)
    h1pad = _compute(
        e1, x3, None, w8_1, wc_1, pw1b, s1, o1,
        n_out_rows=n + B, nblk=nblk, grid=nblk + 1,
        relu_center=True, relu_out=True, add_residual=False,
        out_dtype=jnp.bfloat16)

    h1pad3 = h1pad.reshape(n + B, 2, 128)
    e2 = _scatter(slot_tbl, h1pad3, nblk, relu_src=False)
    out = _compute(
        e2, h1pad3, x, w8_2, wc_2, pw2b, s2, o2,
        n_out_rows=n, nblk=nblk, grid=nblk,
        relu_center=False, relu_out=False, add_residual=True,
        out_dtype=jnp.float32)
    return out


# speedup vs baseline: 1.1274x; 1.1274x over previous
import functools

import jax
import jax.numpy as jnp
from jax.experimental import pallas as pl
from jax.experimental.pallas import tpu as pltpu

EPS = 1e-5
B = 400
S = 8 * B


def _scatter_kernel(slot_hbm, zeros_any, src_blk, ebuf, idx_smem, sbuf, sem_i,
                    sem_s, *, relu_src, dump):
    b = pl.program_id(0)
    cp = pltpu.make_async_copy(slot_hbm.at[b], idx_smem, sem_i)
    cp.start()
    cp.wait()

    v = src_blk[...]
    if relu_src:
        v = jnp.maximum(v, 0.0)
    sbuf[...] = v.astype(jnp.bfloat16)

    def issue(i, carry):
        for kk in range(8):
            d = idx_smem[kk * B + i]
            pltpu.make_async_copy(
                sbuf.at[i],
                ebuf.at[d],
                sem_s,
            ).start(priority=kk % 2)
        return carry

    jax.lax.fori_loop(0, B, issue, 0)
    pltpu.make_async_copy(
        ebuf.at[pl.ds(0, S)], ebuf.at[pl.ds(0, S)], sem_s).wait()


def _scatter(slot_tbl, src3, nblk_src, relu_src):
    ztotal = (nblk_src + 1) * S
    zeros = jnp.zeros((ztotal, 2, 128), jnp.bfloat16)
    kern = functools.partial(_scatter_kernel, relu_src=relu_src,
                             dump=nblk_src * S)
    return pl.pallas_call(
        kern,
        grid=(nblk_src + 1,),
        in_specs=[
            pl.BlockSpec(memory_space=pl.ANY),
            pl.BlockSpec(memory_space=pl.ANY),
            pl.BlockSpec((B, 2, 128), lambda b: (b, 0, 0)),
        ],
        out_specs=pl.BlockSpec(memory_space=pl.ANY),
        out_shape=jax.ShapeDtypeStruct((ztotal, 2, 128), jnp.bfloat16),
        input_output_aliases={1: 0},
        scratch_shapes=[
            pltpu.SMEM((S,), jnp.int32),
            pltpu.VMEM((B, 2, 128), jnp.bfloat16),
            pltpu.SemaphoreType.DMA,
            pltpu.SemaphoreType.DMA,
        ],
        compiler_params=pltpu.CompilerParams(
            dimension_semantics=("parallel",),
        ),
    )(slot_tbl, zeros, src3)


def _compute_kernel(edge_blk, center_blk, res_blk, w8, wc, pw, sc, bi, out,
                    *, nblk, relu_center, relu_out, add_residual, out_dtype):
    b = pl.program_id(0)

    @pl.when(b < nblk)
    def _body():
        ctr = center_blk[...]
        if relu_center:
            ctr = jnp.maximum(ctr, 0.0)
        acc = ctr.astype(jnp.float32) * wc[...]
        for kk in range(8):
            g = edge_blk[kk * B:(kk + 1) * B, :, :].astype(jnp.float32)
            acc = acc + g * w8[kk:kk + 1, :, :]

        l0 = acc[:, 0, :].astype(jnp.bfloat16)
        l1 = acc[:, 1, :].astype(jnp.bfloat16)
        mm = (jnp.dot(l0, pw[:128, :], preferred_element_type=jnp.float32) +
              jnp.dot(l1, pw[128:, :], preferred_element_type=jnp.float32))
        h = mm * sc[...] + bi[...]
        if relu_out:
            h = jnp.maximum(h, 0.0)
        if add_residual:
            h = h + res_blk[...]
        out[...] = h.astype(out_dtype)

    if nblk < pl.num_programs(0):
        @pl.when(b >= nblk)
        def _zero_tail():
            out[...] = jnp.zeros(out.shape, out.dtype)


def _compute(edge_buf, center3, residual, w83, wc3, pw_bf16, sc, bi, *,
             n_out_rows, nblk, grid, relu_center, relu_out, add_residual,
             out_dtype):
    kern = functools.partial(
        _compute_kernel, nblk=nblk, relu_center=relu_center,
        relu_out=relu_out, add_residual=add_residual, out_dtype=out_dtype)
    blk = lambda b: (b, 0)
    blk3 = lambda b: (b, 0, 0)
    zero = lambda b: (0, 0)
    zero3 = lambda b: (0, 0, 0)
    if not add_residual:
        res_spec = pl.BlockSpec((1, 256), zero)
        residual = sc
    else:
        res_spec = pl.BlockSpec((B, 256), blk)
    return pl.pallas_call(
        kern,
        grid=(grid,),
        in_specs=[
            pl.BlockSpec((S, 2, 128), blk3),
            pl.BlockSpec((B, 2, 128), blk3),
            res_spec,
            pl.BlockSpec((8, 2, 128), zero3),
            pl.BlockSpec((1, 2, 128), zero3),
            pl.BlockSpec((256, 256), zero),
            pl.BlockSpec((1, 256), zero),
            pl.BlockSpec((1, 256), zero),
        ],
        out_specs=pl.BlockSpec((B, 256), blk),
        out_shape=jax.ShapeDtypeStruct((n_out_rows, 256), out_dtype),
        compiler_params=pltpu.CompilerParams(
            dimension_semantics=("parallel",),
        ),
    )(edge_buf, center3, residual, w83, wc3, pw_bf16, sc, bi)


def kernel(x, nbr_idx, dw_w1, pw_w1, bn1_g, bn1_b, bn1_m, bn1_v,
           dw_w2, pw_w2, bn2_g, bn2_b, bn2_m, bn2_v):
    n, c = x.shape
    assert c == 256 and n % B == 0
    nblk = n // B
    dump = nblk * S

    idx32 = nbr_idx.astype(jnp.int32)
    sel = jnp.concatenate([idx32[:4], idx32[5:]], axis=0)
    rev = sel[::-1]
    colid = jax.lax.broadcasted_iota(jnp.int32, (8, n), 1)
    dump_slot = dump + (colid % S)
    slot = jnp.where(rev >= 0,
                     (rev // B) * S + jnp.arange(8, dtype=jnp.int32)[:, None] * B
                     + (rev % B),
                     dump_slot)
    slot = jnp.pad(slot, ((0, 0), (0, B)), constant_values=dump)
    slot_tbl = slot.reshape(8, nblk + 1, B).transpose(1, 0, 2)
    slot_tbl = slot_tbl.reshape(nblk + 1, S)

    x3 = x.reshape(n, 2, 128)

    s1 = (bn1_g * jax.lax.rsqrt(bn1_v + EPS)).reshape(1, c)
    o1 = (bn1_b - bn1_m * s1[0]).reshape(1, c)
    s2 = (bn2_g * jax.lax.rsqrt(bn2_v + EPS)).reshape(1, c)
    o2 = (bn2_b - bn2_m * s2[0]).reshape(1, c)

    w8_1 = jnp.concatenate([dw_w1[:4], dw_w1[5:]], axis=0).reshape(8, 2, 128)
    wc_1 = dw_w1[4:5].reshape(1, 2, 128)
    w8_2 = jnp.concatenate([dw_w2[:4], dw_w2[5:]], axis=0).reshape(8, 2, 128)
    wc_2 = dw_w2[4:5].reshape(1, 2, 128)

    pw1b = pw_w1.astype(jnp.bfloat16)
    pw2b = pw_w2.astype(jnp.bfloat16)

    e1 = _scatter(slot_tbl, x3, nblk, relu_src=True)
    h1pad = _compute(
        e1, x3, None, w8_1, wc_1, pw1b, s1, o1,
        n_out_rows=n + B, nblk=nblk, grid=nblk + 1,
        relu_center=True, relu_out=True, add_residual=False,
        out_dtype=jnp.bfloat16)

    h1pad3 = h1pad.reshape(n + B, 2, 128)
    e2 = _scatter(slot_tbl, h1pad3, nblk, relu_src=False)
    out = _compute(
        e2, h1pad3, x, w8_2, wc_2, pw2b, s2, o2,
        n_out_rows=n, nblk=nblk, grid=nblk,
        relu_center=False, relu_out=False, add_residual=True,
        out_dtype=jnp.float32)
    return out


# speedup vs baseline: 1.2349x; 1.0953x over previous
import functools

import jax
import jax.numpy as jnp
from jax.experimental import pallas as pl
from jax.experimental.pallas import tpu as pltpu

EPS = 1e-5
B = 400
S = 8 * B


def _scatter_kernel(slot_hbm, zeros_any, src_blk, ebuf, idx_smem, sbuf, sem_i,
                    sem_s, *, relu_src, steps):
    s = pl.program_id(1)
    p = jax.lax.rem(s, 2)

    cp = pltpu.make_async_copy(slot_hbm.at[pl.program_id(0) * steps + s],
                               idx_smem, sem_i)
    cp.start()
    cp.wait()

    v = src_blk[...]
    if relu_src:
        v = jnp.maximum(v, 0.0)
    sbuf[p] = v.astype(jnp.bfloat16)

    def issue(i, carry):
        for kk in range(8):
            d = idx_smem[kk * B + i]
            pltpu.make_async_copy(
                sbuf.at[p, i],
                ebuf.at[d],
                sem_s.at[p],
            ).start(priority=kk % 2)
        return carry

    jax.lax.fori_loop(0, B, issue, 0)

    @pl.when(s > 0)
    def _wait_prev():
        pltpu.make_async_copy(
            ebuf.at[pl.ds(0, S)], ebuf.at[pl.ds(0, S)], sem_s.at[1 - p]
        ).wait()

    @pl.when(s == steps - 1)
    def _wait_last():
        pltpu.make_async_copy(
            ebuf.at[pl.ds(0, S)], ebuf.at[pl.ds(0, S)], sem_s.at[p]
        ).wait()


def _scatter(slot_tbl, src3, nblk_src, relu_src):
    ztotal = (nblk_src + 1) * S
    steps = (nblk_src + 1) // 2
    assert (nblk_src + 1) % 2 == 0
    zeros = jnp.zeros((ztotal, 2, 128), jnp.bfloat16)
    kern = functools.partial(_scatter_kernel, relu_src=relu_src, steps=steps)
    return pl.pallas_call(
        kern,
        grid=(2, steps),
        in_specs=[
            pl.BlockSpec(memory_space=pl.ANY),
            pl.BlockSpec(memory_space=pl.ANY),
            pl.BlockSpec((B, 2, 128), lambda c, s: (c * steps + s, 0, 0)),
        ],
        out_specs=pl.BlockSpec(memory_space=pl.ANY),
        out_shape=jax.ShapeDtypeStruct((ztotal, 2, 128), jnp.bfloat16),
        input_output_aliases={1: 0},
        scratch_shapes=[
            pltpu.SMEM((S,), jnp.int32),
            pltpu.VMEM((2, B, 2, 128), jnp.bfloat16),
            pltpu.SemaphoreType.DMA,
            pltpu.SemaphoreType.DMA((2,)),
        ],
        compiler_params=pltpu.CompilerParams(
            dimension_semantics=("parallel", "arbitrary"),
        ),
    )(slot_tbl, zeros, src3)


def _compute_kernel(edge_blk, center_blk, res_blk, w8, wc, pw, sc, bi, out,
                    *, nblk, relu_center, relu_out, add_residual, out_dtype):
    b = pl.program_id(0)

    @pl.when(b < nblk)
    def _body():
        ctr = center_blk[...]
        if relu_center:
            ctr = jnp.maximum(ctr, 0.0)
        acc = ctr.astype(jnp.float32) * wc[...]
        for kk in range(8):
            g = edge_blk[kk * B:(kk + 1) * B, :, :].astype(jnp.float32)
            acc = acc + g * w8[kk:kk + 1, :, :]

        l0 = acc[:, 0, :].astype(jnp.bfloat16)
        l1 = acc[:, 1, :].astype(jnp.bfloat16)
        mm = (jnp.dot(l0, pw[:128, :], preferred_element_type=jnp.float32) +
              jnp.dot(l1, pw[128:, :], preferred_element_type=jnp.float32))
        h = mm + bi[...]
        if relu_out:
            h = jnp.maximum(h, 0.0)
        if add_residual:
            h = h + res_blk[...]
        out[...] = h.astype(out_dtype)

    if nblk < pl.num_programs(0):
        @pl.when(b >= nblk)
        def _zero_tail():
            out[...] = jnp.zeros(out.shape, out.dtype)


def _compute(edge_buf, center3, residual, w83, wc3, pw_bf16, sc, bi, *,
             n_out_rows, nblk, grid, relu_center, relu_out, add_residual,
             out_dtype):
    kern = functools.partial(
        _compute_kernel, nblk=nblk, relu_center=relu_center,
        relu_out=relu_out, add_residual=add_residual, out_dtype=out_dtype)
    blk = lambda b: (b, 0)
    blk3 = lambda b: (b, 0, 0)
    zero = lambda b: (0, 0)
    zero3 = lambda b: (0, 0, 0)
    if not add_residual:
        res_spec = pl.BlockSpec((1, 256), zero)
        residual = sc
    else:
        res_spec = pl.BlockSpec((B, 256), blk)
    return pl.pallas_call(
        kern,
        grid=(grid,),
        in_specs=[
            pl.BlockSpec((S, 2, 128), blk3),
            pl.BlockSpec((B, 2, 128), blk3),
            res_spec,
            pl.BlockSpec((8, 2, 128), zero3),
            pl.BlockSpec((1, 2, 128), zero3),
            pl.BlockSpec((256, 256), zero),
            pl.BlockSpec((1, 256), zero),
            pl.BlockSpec((1, 256), zero),
        ],
        out_specs=pl.BlockSpec((B, 256), blk),
        out_shape=jax.ShapeDtypeStruct((n_out_rows, 256), out_dtype),
        compiler_params=pltpu.CompilerParams(
            dimension_semantics=("parallel",),
        ),
    )(edge_buf, center3, residual, w83, wc3, pw_bf16, sc, bi)


def kernel(x, nbr_idx, dw_w1, pw_w1, bn1_g, bn1_b, bn1_m, bn1_v,
           dw_w2, pw_w2, bn2_g, bn2_b, bn2_m, bn2_v):
    n, c = x.shape
    assert c == 256 and n % B == 0
    nblk = n // B
    dump = nblk * S

    idx32 = nbr_idx.astype(jnp.int32)
    sel = jnp.concatenate([idx32[:4], idx32[5:]], axis=0)
    rev = sel[::-1]
    colid = jax.lax.broadcasted_iota(jnp.int32, (8, n), 1)
    dump_slot = dump + (colid % S)
    slot = jnp.where(rev >= 0,
                     (rev // B) * S + jnp.arange(8, dtype=jnp.int32)[:, None] * B
                     + (rev % B),
                     dump_slot)
    slot = jnp.pad(slot, ((0, 0), (0, B)), constant_values=dump)
    slot_tbl = slot.reshape(8, nblk + 1, B).transpose(1, 0, 2)
    slot_tbl = slot_tbl.reshape(nblk + 1, S)

    x3 = x.reshape(n, 2, 128)

    s1 = (bn1_g * jax.lax.rsqrt(bn1_v + EPS)).reshape(1, c)
    o1 = (bn1_b - bn1_m * s1[0]).reshape(1, c)
    s2 = (bn2_g * jax.lax.rsqrt(bn2_v + EPS)).reshape(1, c)
    o2 = (bn2_b - bn2_m * s2[0]).reshape(1, c)

    w8_1 = jnp.concatenate([dw_w1[:4], dw_w1[5:]], axis=0).reshape(8, 2, 128)
    wc_1 = dw_w1[4:5].reshape(1, 2, 128)
    w8_2 = jnp.concatenate([dw_w2[:4], dw_w2[5:]], axis=0).reshape(8, 2, 128)
    wc_2 = dw_w2[4:5].reshape(1, 2, 128)

    pw1b = (pw_w1 * s1).astype(jnp.bfloat16)
    pw2b = (pw_w2 * s2).astype(jnp.bfloat16)

    e1 = _scatter(slot_tbl, x3, nblk, relu_src=True)
    h1pad = _compute(
        e1, x3, None, w8_1, wc_1, pw1b, s1, o1,
        n_out_rows=n + B, nblk=nblk, grid=nblk + 1,
        relu_center=True, relu_out=True, add_residual=False,
        out_dtype=jnp.bfloat16)

    h1pad3 = h1pad.reshape(n + B, 2, 128)
    e2 = _scatter(slot_tbl, h1pad3, nblk, relu_src=False)
    out = _compute(
        e2, h1pad3, x, w8_2, wc_2, pw2b, s2, o2,
        n_out_rows=n, nblk=nblk, grid=nblk,
        relu_center=False, relu_out=False, add_residual=True,
        out_dtype=jnp.float32)
    return out
